# Initial kernel scaffold; baseline (speedup 1.0000x reference)
import jax
import jax.numpy as jnp
from jax.experimental import pallas as pl


def kernel(x_drug, x_protein, edge_index_ddi, edge_index_targets, edge_index_rev_targets, params):
    raise NotImplementedError("write your pallas kernel here")


# speedup vs baseline: 1.3231x; 1.3231x over previous
import functools
import jax
import jax.numpy as jnp
from jax import lax
from jax.experimental import pallas as pl
from jax.experimental.pallas import tpu as pltpu
from jax.experimental.pallas import tpu_sc as plsc

NC = 2
NS = 16
NW = NC * NS
K = 128
SB = 16
ZB = 64


def _round_up(n, m):
    return (n + m - 1) // m * m


def _make_agg(n_src_pad, n_dst_pad, S, Wc, n_chunks):
    rows_per_tile = n_dst_pad // NS
    nz = rows_per_tile // ZB
    nb = S // SB
    mesh = plsc.VectorSubcoreMesh(core_axis_name="c", subcore_axis_name="s")
    out_type = [jax.ShapeDtypeStruct((NC, n_dst_pad, Wc), jnp.float32)
                for _ in range(n_chunks)]
    scratch = [
        pltpu.VMEM((SB, K), jnp.int32),
        pltpu.VMEM((SB, K), jnp.int32),
        pltpu.VMEM((K, Wc), jnp.float32),
        pltpu.VMEM((ZB, Wc), jnp.float32),
        pltpu.SemaphoreType.DMA,
        pltpu.VMEM_SHARED((n_dst_pad, Wc), jnp.float32),
    ]

    def body(*refs):
        xs = refs[:n_chunks]
        src_hbm = refs[n_chunks]
        dst_hbm = refs[n_chunks + 1]
        outs = refs[n_chunks + 2: 2 * n_chunks + 2]
        src_v, dst_v, rows_v, zero_v, sem, acc = refs[2 * n_chunks + 2:]
        cid = lax.axis_index("c")
        sid = lax.axis_index("s")
        w = cid * NS + sid

        zeros16 = jnp.zeros((16,), jnp.float32)

        def zfill(i, carry):
            for k in range(Wc // 16):
                zero_v[i, pl.ds(k * 16, 16)] = zeros16
            return carry
        lax.fori_loop(0, ZB, zfill, 0)

        tbase = sid * rows_per_tile
        for c in range(n_chunks):
            def zcopy(i, carry):
                pltpu.sync_copy(zero_v, acc.at[pl.ds(tbase + i * ZB, ZB)])
                return carry
            lax.fori_loop(0, nz, zcopy, 0)
            plsc.subcore_barrier()

            x_hbm = xs[c]

            def blk(bi, carry):
                pltpu.sync_copy(src_hbm.at[w, pl.ds(bi * SB, SB)], src_v)
                pltpu.sync_copy(dst_hbm.at[w, pl.ds(bi * SB, SB)], dst_v)

                def step(s, carry2):
                    pltpu.async_copy(x_hbm.at[src_v.at[s]], rows_v,
                                     sem).wait()
                    pltpu.sync_copy(rows_v, acc.at[dst_v.at[s]], add=True)
                    return carry2
                lax.fori_loop(0, SB, step, 0)
                return carry
            lax.fori_loop(0, nb, blk, 0)
            plsc.subcore_barrier()

            pltpu.sync_copy(acc.at[pl.ds(tbase, rows_per_tile)],
                            outs[c].at[cid, pl.ds(tbase, rows_per_tile)])

    return pl.kernel(body, out_type=out_type, mesh=mesh, scratch_types=scratch,
                     compiler_params=pltpu.CompilerParams(
                         use_tc_tiling_on_sc=False))


def _make_cnt(n_dst_pad, S):
    Wc = 16
    rows_per_tile = n_dst_pad // NS
    nz = rows_per_tile // ZB
    nb = S // SB
    mesh = plsc.VectorSubcoreMesh(core_axis_name="c", subcore_axis_name="s")
    out_type = jax.ShapeDtypeStruct((NC, n_dst_pad, Wc), jnp.float32)
    scratch = [
        pltpu.VMEM((SB, K), jnp.int32),
        pltpu.VMEM((K, Wc), jnp.float32),
        pltpu.VMEM((ZB, Wc), jnp.float32),
        pltpu.VMEM_SHARED((n_dst_pad, Wc), jnp.float32),
    ]

    def body(dst_hbm, out_hbm, dst_v, ones_v, zero_v, acc):
        cid = lax.axis_index("c")
        sid = lax.axis_index("s")
        w = cid * NS + sid

        ones16 = jnp.ones((16,), jnp.float32)
        zeros16 = jnp.zeros((16,), jnp.float32)

        def fill(i, carry):
            zero_v[i % ZB, pl.ds(0, 16)] = zeros16
            ones_v[i, pl.ds(0, 16)] = ones16
            return carry
        lax.fori_loop(0, K, fill, 0)

        tbase = sid * rows_per_tile

        def zcopy(i, carry):
            pltpu.sync_copy(zero_v, acc.at[pl.ds(tbase + i * ZB, ZB)])
            return carry
        lax.fori_loop(0, nz, zcopy, 0)
        plsc.subcore_barrier()

        def blk(bi, carry):
            pltpu.sync_copy(dst_hbm.at[w, pl.ds(bi * SB, SB)], dst_v)

            def step(s, carry2):
                pltpu.sync_copy(ones_v, acc.at[dst_v.at[s]], add=True)
                return carry2
            lax.fori_loop(0, SB, step, 0)
            return carry
        lax.fori_loop(0, nb, blk, 0)
        plsc.subcore_barrier()

        pltpu.sync_copy(acc.at[pl.ds(tbase, rows_per_tile)],
                        out_hbm.at[cid, pl.ds(tbase, rows_per_tile)])

    return pl.kernel(body, out_type=out_type, mesh=mesh, scratch_types=scratch,
                     compiler_params=pltpu.CompilerParams(
                         use_tc_tiling_on_sc=False))


def _make_update(n_pad, blk, d_dst, d_out, relu, chunk_lists):
    grid = (n_pad // blk,)
    in_specs = []
    for (n_chunks, Wc) in chunk_lists:
        for _ in range(n_chunks):
            in_specs.append(pl.BlockSpec((NC, blk, Wc), lambda i: (0, i, 0)))
        in_specs.append(pl.BlockSpec((NC, blk, 16), lambda i: (0, i, 0)))
        for _ in range(n_chunks):
            in_specs.append(pl.BlockSpec((Wc, d_out), lambda i: (0, 0)))
    in_specs.append(pl.BlockSpec((blk, d_dst), lambda i: (i, 0)))
    in_specs.append(pl.BlockSpec((d_dst, d_out), lambda i: (0, 0)))
    in_specs.append(pl.BlockSpec((8, d_out), lambda i: (0, 0)))
    out_specs = pl.BlockSpec((blk, d_out), lambda i: (i, 0))

    def body(*refs):
        idx = 0
        acc = None
        for (n_chunks, Wc) in chunk_lists:
            s_refs = refs[idx:idx + n_chunks]; idx += n_chunks
            cnt_ref = refs[idx]; idx += 1
            wl_refs = refs[idx:idx + n_chunks]; idx += n_chunks
            cnt = cnt_ref[0, :, 0:1] + cnt_ref[1, :, 0:1]
            inv = 1.0 / jnp.maximum(cnt, 1.0)
            for s_ref, wl_ref in zip(s_refs, wl_refs):
                mean = (s_ref[0] + s_ref[1]) * inv
                d = jnp.dot(mean, wl_ref[...],
                            preferred_element_type=jnp.float32)
                acc = d if acc is None else acc + d
        x_ref = refs[idx]
        wr_ref = refs[idx + 1]
        b_ref = refs[idx + 2]
        o_ref = refs[idx + 3]
        acc = acc + jnp.dot(x_ref[...], wr_ref[...],
                            preferred_element_type=jnp.float32) + b_ref[0:1, :]
        if relu:
            acc = jnp.maximum(acc, 0.0)
        o_ref[...] = acc

    return pl.pallas_call(
        body, grid=grid, in_specs=in_specs, out_specs=out_specs,
        out_shape=jax.ShapeDtypeStruct((n_pad, d_out), jnp.float32))


def _prep_edges(ei, n_dst):
    e = ei.shape[1]
    s_steps = _round_up(_round_up(e, NW * K) // (NW * K), SB)
    e_pad = NW * s_steps * K
    src = jnp.pad(ei[0], (0, e_pad - e), constant_values=0)
    dst = jnp.pad(ei[1], (0, e_pad - e), constant_values=n_dst)
    return (src.reshape(NW, s_steps, K), dst.reshape(NW, s_steps, K), s_steps)


def _chunks(x, n_chunks):
    wc = x.shape[1] // n_chunks
    return [x[:, c * wc:(c + 1) * wc] for c in range(n_chunks)]


def _row_chunks(x, n_chunks):
    wc = x.shape[0] // n_chunks
    return [x[c * wc:(c + 1) * wc, :] for c in range(n_chunks)]


def kernel(x_drug, x_protein, edge_index_ddi, edge_index_targets,
           edge_index_rev_targets, params):
    p = params
    nd, din = x_drug.shape
    npr, pin = x_protein.shape
    h = p['l1_ddi_Wl'].shape[0]
    d_out = p['l3_ddi_Wl'].shape[0]

    nd_pad = _round_up(nd, NS * ZB * 2)
    np_pad = _round_up(npr, NS * ZB * 2)
    xd = jnp.pad(x_drug, ((0, nd_pad - nd), (0, 0)))
    xp = jnp.pad(x_protein, ((0, np_pad - npr), (0, 0)))

    src_ddi, dst_ddi, s_ddi = _prep_edges(edge_index_ddi, nd)
    src_t, dst_t, s_t = _prep_edges(edge_index_targets, npr)
    src_rt, dst_rt, s_rt = _prep_edges(edge_index_rev_targets, nd)

    cnt_ddi_k = _make_cnt(nd_pad, s_ddi)
    cnt_t_k = _make_cnt(np_pad, s_t)
    cnt_rt_k = _make_cnt(nd_pad, s_rt)
    agg_ddi_k = _make_agg(nd_pad, nd_pad, s_ddi, 32, 4)
    agg_rt1_k = _make_agg(np_pad, nd_pad, s_rt, pin, 1)
    agg_rt_k = _make_agg(np_pad, nd_pad, s_rt, 32, 4)
    agg_t_k = _make_agg(nd_pad, np_pad, s_t, h, 1)

    cnt_ddi = cnt_ddi_k(dst_ddi)
    cnt_t = cnt_t_k(dst_t)
    cnt_rt = cnt_rt_k(dst_rt)

    blk = 1024
    upd_d1 = _make_update(nd_pad, blk, din, h, True, [(4, 32), (1, pin)])
    upd_p1 = _make_update(np_pad, blk, pin, h, True, [(1, h)])
    upd_d2 = _make_update(nd_pad, blk, h, h, True, [(4, 32), (4, 32)])
    upd_p2 = _make_update(np_pad, blk, h, h, True, [(1, h)])
    upd_d3 = _make_update(nd_pad, blk, h, d_out, False, [(4, 32), (4, 32)])
    upd_p3 = _make_update(np_pad, blk, h, d_out, False, [(1, h)])

    def wT(name):
        return p[name].T

    def bias8(b):
        return jnp.broadcast_to(b.reshape(1, -1), (8, b.shape[0]))

    def as_list(x):
        return list(x) if isinstance(x, (list, tuple)) else [x]

    def drug_layer(upd, hd, hp, pre, agg_rt_kern, rt_chunks):
        s_ddi_parts = as_list(agg_ddi_k(*_chunks(hd, 4), src_ddi, dst_ddi))
        s_rt_parts = as_list(agg_rt_kern(*_chunks(hp, rt_chunks),
                                         src_rt, dst_rt))
        wl_ddi = _row_chunks(wT(pre + '_ddi_Wl'), 4)
        wl_rt = _row_chunks(wT(pre + '_rt_Wl'), rt_chunks)
        wr = wT(pre + '_ddi_Wr') + wT(pre + '_rt_Wr')
        b = bias8(p[pre + '_ddi_bl'] + p[pre + '_rt_bl'])
        return upd(*s_ddi_parts, cnt_ddi, *wl_ddi,
                   *s_rt_parts, cnt_rt, *wl_rt, hd, wr, b)

    def prot_layer(upd, hd, hp, pre):
        s_t_parts = as_list(agg_t_k(hd, src_t, dst_t))
        return upd(s_t_parts[0], cnt_t, wT(pre + '_t_Wl'), hp,
                   wT(pre + '_t_Wr'), bias8(p[pre + '_t_bl']))

    hd1 = drug_layer(upd_d1, xd, xp, 'l1', agg_rt1_k, 1)
    hp1 = prot_layer(upd_p1, xd, xp, 'l1')
    hd2 = drug_layer(upd_d2, hd1, hp1, 'l2', agg_rt_k, 4)
    hp2 = prot_layer(upd_p2, hd1, hp1, 'l2')
    od = drug_layer(upd_d3, hd2, hp2, 'l3', agg_rt_k, 4)
    op = prot_layer(upd_p3, hd2, hp2, 'l3')

    return od[:nd], op[:npr]


# speedup vs baseline: 1.4442x; 1.0916x over previous
import functools
import jax
import jax.numpy as jnp
from jax import lax
from jax.experimental import pallas as pl
from jax.experimental.pallas import tpu as pltpu
from jax.experimental.pallas import tpu_sc as plsc

NC = 2
NS = 16
NW = NC * NS
K = 128
SB = 16
ZB = 64


def _round_up(n, m):
    return (n + m - 1) // m * m


def _make_agg(n_src_pad, n_dst_pad, S, Wc, n_chunks):
    rows_per_tile = n_dst_pad // NS
    nz = rows_per_tile // ZB
    nb = S // SB
    nbuf = 4
    mesh = plsc.VectorSubcoreMesh(core_axis_name="c", subcore_axis_name="s")
    out_type = [jax.ShapeDtypeStruct((NC, n_dst_pad, Wc), jnp.float32)
                for _ in range(n_chunks)]
    scratch = [
        pltpu.VMEM((SB, K), jnp.int32),
        pltpu.VMEM((SB, K), jnp.int32),
        pltpu.VMEM((nbuf, K, Wc), jnp.float32),
        pltpu.VMEM((ZB, Wc), jnp.float32),
    ] + [pltpu.SemaphoreType.DMA] * nbuf + [
        pltpu.VMEM_SHARED((n_dst_pad, Wc), jnp.float32),
    ]

    def body(*refs):
        xs = refs[:n_chunks]
        src_hbm = refs[n_chunks]
        dst_hbm = refs[n_chunks + 1]
        outs = refs[n_chunks + 2: 2 * n_chunks + 2]
        rest = refs[2 * n_chunks + 2:]
        src_v, dst_v, rows_v, zero_v = rest[:4]
        sems = rest[4:4 + nbuf]
        acc = rest[4 + nbuf]
        cid = lax.axis_index("c")
        sid = lax.axis_index("s")
        w = cid * NS + sid

        zeros16 = jnp.zeros((16,), jnp.float32)

        def zfill(i, carry):
            for k in range(Wc // 16):
                zero_v[i, pl.ds(k * 16, 16)] = zeros16
            return carry
        lax.fori_loop(0, ZB, zfill, 0)

        tbase = sid * rows_per_tile
        for c in range(n_chunks):
            def zcopy(i, carry):
                pltpu.sync_copy(zero_v, acc.at[pl.ds(tbase + i * ZB, ZB)])
                return carry
            lax.fori_loop(0, nz, zcopy, 0)
            plsc.subcore_barrier()

            x_hbm = xs[c]

            def blk(bi, carry):
                pltpu.sync_copy(src_hbm.at[w, pl.ds(bi * SB, SB)], src_v)
                pltpu.sync_copy(dst_hbm.at[w, pl.ds(bi * SB, SB)], dst_v)
                descs = [
                    pltpu.async_copy(x_hbm.at[src_v.at[j]], rows_v.at[j],
                                     sems[j])
                    for j in range(nbuf)
                ]
                for s in range(SB):
                    j = s % nbuf
                    descs[j].wait()
                    pltpu.sync_copy(rows_v.at[j], acc.at[dst_v.at[s]],
                                    add=True)
                    if s + nbuf < SB:
                        descs[j] = pltpu.async_copy(
                            x_hbm.at[src_v.at[s + nbuf]], rows_v.at[j],
                            sems[j])
                return carry
            lax.fori_loop(0, nb, blk, 0)
            plsc.subcore_barrier()

            pltpu.sync_copy(acc.at[pl.ds(tbase, rows_per_tile)],
                            outs[c].at[cid, pl.ds(tbase, rows_per_tile)])

    return pl.kernel(body, out_type=out_type, mesh=mesh, scratch_types=scratch,
                     compiler_params=pltpu.CompilerParams(
                         use_tc_tiling_on_sc=False))


def _make_cnt(n_dst_pad, S):
    Wc = 16
    rows_per_tile = n_dst_pad // NS
    nz = rows_per_tile // ZB
    nb = S // SB
    mesh = plsc.VectorSubcoreMesh(core_axis_name="c", subcore_axis_name="s")
    out_type = jax.ShapeDtypeStruct((NC, n_dst_pad, Wc), jnp.float32)
    scratch = [
        pltpu.VMEM((SB, K), jnp.int32),
        pltpu.VMEM((K, Wc), jnp.float32),
        pltpu.VMEM((ZB, Wc), jnp.float32),
        pltpu.VMEM_SHARED((n_dst_pad, Wc), jnp.float32),
    ]

    def body(dst_hbm, out_hbm, dst_v, ones_v, zero_v, acc):
        cid = lax.axis_index("c")
        sid = lax.axis_index("s")
        w = cid * NS + sid

        ones16 = jnp.ones((16,), jnp.float32)
        zeros16 = jnp.zeros((16,), jnp.float32)

        def fill(i, carry):
            zero_v[i % ZB, pl.ds(0, 16)] = zeros16
            ones_v[i, pl.ds(0, 16)] = ones16
            return carry
        lax.fori_loop(0, K, fill, 0)

        tbase = sid * rows_per_tile

        def zcopy(i, carry):
            pltpu.sync_copy(zero_v, acc.at[pl.ds(tbase + i * ZB, ZB)])
            return carry
        lax.fori_loop(0, nz, zcopy, 0)
        plsc.subcore_barrier()

        def blk(bi, carry):
            pltpu.sync_copy(dst_hbm.at[w, pl.ds(bi * SB, SB)], dst_v)

            def step(s, carry2):
                pltpu.sync_copy(ones_v, acc.at[dst_v.at[s]], add=True)
                return carry2
            lax.fori_loop(0, SB, step, 0)
            return carry
        lax.fori_loop(0, nb, blk, 0)
        plsc.subcore_barrier()

        pltpu.sync_copy(acc.at[pl.ds(tbase, rows_per_tile)],
                        out_hbm.at[cid, pl.ds(tbase, rows_per_tile)])

    return pl.kernel(body, out_type=out_type, mesh=mesh, scratch_types=scratch,
                     compiler_params=pltpu.CompilerParams(
                         use_tc_tiling_on_sc=False))


def _make_update(n_pad, blk, d_dst, d_out, relu, chunk_lists):
    grid = (n_pad // blk,)
    in_specs = []
    for (n_chunks, Wc) in chunk_lists:
        for _ in range(n_chunks):
            in_specs.append(pl.BlockSpec((NC, blk, Wc), lambda i: (0, i, 0)))
        in_specs.append(pl.BlockSpec((NC, blk, 16), lambda i: (0, i, 0)))
        for _ in range(n_chunks):
            in_specs.append(pl.BlockSpec((Wc, d_out), lambda i: (0, 0)))
    in_specs.append(pl.BlockSpec((blk, d_dst), lambda i: (i, 0)))
    in_specs.append(pl.BlockSpec((d_dst, d_out), lambda i: (0, 0)))
    in_specs.append(pl.BlockSpec((8, d_out), lambda i: (0, 0)))
    out_specs = pl.BlockSpec((blk, d_out), lambda i: (i, 0))

    def body(*refs):
        idx = 0
        acc = None
        for (n_chunks, Wc) in chunk_lists:
            s_refs = refs[idx:idx + n_chunks]; idx += n_chunks
            cnt_ref = refs[idx]; idx += 1
            wl_refs = refs[idx:idx + n_chunks]; idx += n_chunks
            cnt = cnt_ref[0, :, 0:1] + cnt_ref[1, :, 0:1]
            inv = 1.0 / jnp.maximum(cnt, 1.0)
            for s_ref, wl_ref in zip(s_refs, wl_refs):
                mean = (s_ref[0] + s_ref[1]) * inv
                d = jnp.dot(mean, wl_ref[...],
                            preferred_element_type=jnp.float32)
                acc = d if acc is None else acc + d
        x_ref = refs[idx]
        wr_ref = refs[idx + 1]
        b_ref = refs[idx + 2]
        o_ref = refs[idx + 3]
        acc = acc + jnp.dot(x_ref[...], wr_ref[...],
                            preferred_element_type=jnp.float32) + b_ref[0:1, :]
        if relu:
            acc = jnp.maximum(acc, 0.0)
        o_ref[...] = acc

    return pl.pallas_call(
        body, grid=grid, in_specs=in_specs, out_specs=out_specs,
        out_shape=jax.ShapeDtypeStruct((n_pad, d_out), jnp.float32))


def _prep_edges(ei, n_dst):
    e = ei.shape[1]
    s_steps = _round_up(_round_up(e, NW * K) // (NW * K), SB)
    e_pad = NW * s_steps * K
    src = jnp.pad(ei[0], (0, e_pad - e), constant_values=0)
    dst = jnp.pad(ei[1], (0, e_pad - e), constant_values=n_dst)
    return (src.reshape(NW, s_steps, K), dst.reshape(NW, s_steps, K), s_steps)


def _chunks(x, n_chunks):
    wc = x.shape[1] // n_chunks
    return [x[:, c * wc:(c + 1) * wc] for c in range(n_chunks)]


def _row_chunks(x, n_chunks):
    wc = x.shape[0] // n_chunks
    return [x[c * wc:(c + 1) * wc, :] for c in range(n_chunks)]


def kernel(x_drug, x_protein, edge_index_ddi, edge_index_targets,
           edge_index_rev_targets, params):
    p = params
    nd, din = x_drug.shape
    npr, pin = x_protein.shape
    h = p['l1_ddi_Wl'].shape[0]
    d_out = p['l3_ddi_Wl'].shape[0]

    nd_pad = _round_up(nd, NS * ZB * 2)
    np_pad = _round_up(npr, NS * ZB * 2)
    xd = jnp.pad(x_drug, ((0, nd_pad - nd), (0, 0)))
    xp = jnp.pad(x_protein, ((0, np_pad - npr), (0, 0)))

    src_ddi, dst_ddi, s_ddi = _prep_edges(edge_index_ddi, nd)
    src_t, dst_t, s_t = _prep_edges(edge_index_targets, npr)
    src_rt, dst_rt, s_rt = _prep_edges(edge_index_rev_targets, nd)

    cnt_ddi_k = _make_cnt(nd_pad, s_ddi)
    cnt_t_k = _make_cnt(np_pad, s_t)
    cnt_rt_k = _make_cnt(nd_pad, s_rt)
    agg_ddi_k = _make_agg(nd_pad, nd_pad, s_ddi, 32, 4)
    agg_rt1_k = _make_agg(np_pad, nd_pad, s_rt, pin, 1)
    agg_rt_k = _make_agg(np_pad, nd_pad, s_rt, 32, 4)
    agg_t_k = _make_agg(nd_pad, np_pad, s_t, 64, 2)

    cnt_ddi = cnt_ddi_k(dst_ddi)
    cnt_t = cnt_t_k(dst_t)
    cnt_rt = cnt_rt_k(dst_rt)

    blk = 1024
    upd_d1 = _make_update(nd_pad, blk, din, h, True, [(4, 32), (1, pin)])
    upd_p1 = _make_update(np_pad, blk, pin, h, True, [(2, 64)])
    upd_d2 = _make_update(nd_pad, blk, h, h, True, [(4, 32), (4, 32)])
    upd_p2 = _make_update(np_pad, blk, h, h, True, [(2, 64)])
    upd_d3 = _make_update(nd_pad, blk, h, d_out, False, [(4, 32), (4, 32)])
    upd_p3 = _make_update(np_pad, blk, h, d_out, False, [(2, 64)])

    def wT(name):
        return p[name].T

    def bias8(b):
        return jnp.broadcast_to(b.reshape(1, -1), (8, b.shape[0]))

    def as_list(x):
        return list(x) if isinstance(x, (list, tuple)) else [x]

    def drug_layer(upd, hd, hp, pre, agg_rt_kern, rt_chunks):
        s_ddi_parts = as_list(agg_ddi_k(*_chunks(hd, 4), src_ddi, dst_ddi))
        s_rt_parts = as_list(agg_rt_kern(*_chunks(hp, rt_chunks),
                                         src_rt, dst_rt))
        wl_ddi = _row_chunks(wT(pre + '_ddi_Wl'), 4)
        wl_rt = _row_chunks(wT(pre + '_rt_Wl'), rt_chunks)
        wr = wT(pre + '_ddi_Wr') + wT(pre + '_rt_Wr')
        b = bias8(p[pre + '_ddi_bl'] + p[pre + '_rt_bl'])
        return upd(*s_ddi_parts, cnt_ddi, *wl_ddi,
                   *s_rt_parts, cnt_rt, *wl_rt, hd, wr, b)

    def prot_layer(upd, hd, hp, pre):
        s_t_parts = as_list(agg_t_k(*_chunks(hd, 2), src_t, dst_t))
        return upd(*s_t_parts, cnt_t, *_row_chunks(wT(pre + '_t_Wl'), 2), hp,
                   wT(pre + '_t_Wr'), bias8(p[pre + '_t_bl']))

    hd1 = drug_layer(upd_d1, xd, xp, 'l1', agg_rt1_k, 1)
    hp1 = prot_layer(upd_p1, xd, xp, 'l1')
    hd2 = drug_layer(upd_d2, hd1, hp1, 'l2', agg_rt_k, 4)
    hp2 = prot_layer(upd_p2, hd1, hp1, 'l2')
    od = drug_layer(upd_d3, hd2, hp2, 'l3', agg_rt_k, 4)
    op = prot_layer(upd_p3, hd2, hp2, 'l3')

    return od[:nd], op[:npr]


# speedup vs baseline: 1.7340x; 1.2007x over previous
import functools
import jax
import jax.numpy as jnp
from jax import lax
from jax.experimental import pallas as pl
from jax.experimental.pallas import tpu as pltpu
from jax.experimental.pallas import tpu_sc as plsc

NC = 2
NS = 16
NW = NC * NS
K = 128
SB = 16
ZB = 64


def _round_up(n, m):
    return (n + m - 1) // m * m


def _make_agg(n_src_pad, n_dst_pad, S, Wc, n_chunks):
    rows_per_tile = n_dst_pad // NS
    nz = rows_per_tile // ZB
    nb = S // SB
    nbuf = 4
    mesh = plsc.VectorSubcoreMesh(core_axis_name="c", subcore_axis_name="s")
    out_type = [jax.ShapeDtypeStruct((NC, n_dst_pad, Wc), jnp.float32)
                for _ in range(n_chunks)]
    scratch = [
        pltpu.VMEM((SB, K), jnp.int32),
        pltpu.VMEM((SB, K), jnp.int32),
        pltpu.VMEM((nbuf, K, Wc), jnp.float32),
        pltpu.VMEM((ZB, Wc), jnp.float32),
    ] + [pltpu.SemaphoreType.DMA] * nbuf + [
        pltpu.VMEM_SHARED((n_dst_pad, Wc), jnp.float32),
    ]

    def body(*refs):
        xs = refs[:n_chunks]
        src_hbm = refs[n_chunks]
        dst_hbm = refs[n_chunks + 1]
        outs = refs[n_chunks + 2: 2 * n_chunks + 2]
        rest = refs[2 * n_chunks + 2:]
        src_v, dst_v, rows_v, zero_v = rest[:4]
        sems = rest[4:4 + nbuf]
        acc = rest[4 + nbuf]
        cid = lax.axis_index("c")
        sid = lax.axis_index("s")
        w = cid * NS + sid

        zeros16 = jnp.zeros((16,), jnp.float32)

        def zfill(i, carry):
            for k in range(Wc // 16):
                zero_v[i, pl.ds(k * 16, 16)] = zeros16
            return carry
        lax.fori_loop(0, ZB, zfill, 0)

        tbase = sid * rows_per_tile
        for c in range(n_chunks):
            def zcopy(i, carry):
                pltpu.sync_copy(zero_v, acc.at[pl.ds(tbase + i * ZB, ZB)])
                return carry
            lax.fori_loop(0, nz, zcopy, 0)
            plsc.subcore_barrier()

            x_hbm = xs[c]

            def blk(bi, carry):
                pltpu.sync_copy(src_hbm.at[w, pl.ds(bi * SB, SB)], src_v)
                pltpu.sync_copy(dst_hbm.at[w, pl.ds(bi * SB, SB)], dst_v)
                descs = [
                    pltpu.async_copy(x_hbm.at[src_v.at[j]], rows_v.at[j],
                                     sems[j])
                    for j in range(nbuf)
                ]
                for s in range(SB):
                    j = s % nbuf
                    descs[j].wait()
                    pltpu.sync_copy(rows_v.at[j], acc.at[dst_v.at[s]],
                                    add=True)
                    if s + nbuf < SB:
                        descs[j] = pltpu.async_copy(
                            x_hbm.at[src_v.at[s + nbuf]], rows_v.at[j],
                            sems[j])
                return carry
            lax.fori_loop(0, nb, blk, 0)
            plsc.subcore_barrier()

            pltpu.sync_copy(acc.at[pl.ds(tbase, rows_per_tile)],
                            outs[c].at[cid, pl.ds(tbase, rows_per_tile)])

    return pl.kernel(body, out_type=out_type, mesh=mesh, scratch_types=scratch,
                     compiler_params=pltpu.CompilerParams(
                         use_tc_tiling_on_sc=False))


def _make_cnt(n_dst_pad, S):
    Wc = 16
    rows_per_tile = n_dst_pad // NS
    nz = rows_per_tile // ZB
    nb = S // SB
    mesh = plsc.VectorSubcoreMesh(core_axis_name="c", subcore_axis_name="s")
    out_type = jax.ShapeDtypeStruct((NC, n_dst_pad, Wc), jnp.float32)
    scratch = [
        pltpu.VMEM((SB, K), jnp.int32),
        pltpu.VMEM((K, Wc), jnp.float32),
        pltpu.VMEM((ZB, Wc), jnp.float32),
        pltpu.VMEM_SHARED((n_dst_pad, Wc), jnp.float32),
    ]

    def body(dst_hbm, out_hbm, dst_v, ones_v, zero_v, acc):
        cid = lax.axis_index("c")
        sid = lax.axis_index("s")
        w = cid * NS + sid

        ones16 = jnp.ones((16,), jnp.float32)
        zeros16 = jnp.zeros((16,), jnp.float32)

        def fill(i, carry):
            zero_v[i % ZB, pl.ds(0, 16)] = zeros16
            ones_v[i, pl.ds(0, 16)] = ones16
            return carry
        lax.fori_loop(0, K, fill, 0)

        tbase = sid * rows_per_tile

        def zcopy(i, carry):
            pltpu.sync_copy(zero_v, acc.at[pl.ds(tbase + i * ZB, ZB)])
            return carry
        lax.fori_loop(0, nz, zcopy, 0)
        plsc.subcore_barrier()

        def blk(bi, carry):
            pltpu.sync_copy(dst_hbm.at[w, pl.ds(bi * SB, SB)], dst_v)

            def step(s, carry2):
                pltpu.sync_copy(ones_v, acc.at[dst_v.at[s]], add=True)
                return carry2
            lax.fori_loop(0, SB, step, 0)
            return carry
        lax.fori_loop(0, nb, blk, 0)
        plsc.subcore_barrier()

        pltpu.sync_copy(acc.at[pl.ds(tbase, rows_per_tile)],
                        out_hbm.at[cid, pl.ds(tbase, rows_per_tile)])

    return pl.kernel(body, out_type=out_type, mesh=mesh, scratch_types=scratch,
                     compiler_params=pltpu.CompilerParams(
                         use_tc_tiling_on_sc=False))


def _make_matmul(n_pad, blk, d_in, d_out):
    def body(x_ref, w_ref, o_ref):
        o_ref[...] = jnp.dot(x_ref[...], w_ref[...],
                             preferred_element_type=jnp.float32)

    return pl.pallas_call(
        body, grid=(n_pad // blk,),
        in_specs=[pl.BlockSpec((blk, d_in), lambda i: (i, 0)),
                  pl.BlockSpec((d_in, d_out), lambda i: (0, 0))],
        out_specs=pl.BlockSpec((blk, d_out), lambda i: (i, 0)),
        out_shape=jax.ShapeDtypeStruct((n_pad, d_out), jnp.float32))


def _make_update(n_pad, blk, d_dst, d_out, relu, chunk_lists):
    grid = (n_pad // blk,)
    in_specs = []
    for (n_chunks, Wc) in chunk_lists:
        for _ in range(n_chunks):
            in_specs.append(pl.BlockSpec((NC, blk, Wc), lambda i: (0, i, 0)))
        in_specs.append(pl.BlockSpec((NC, blk, 16), lambda i: (0, i, 0)))
        for _ in range(n_chunks):
            in_specs.append(pl.BlockSpec((Wc, d_out), lambda i: (0, 0)))
    in_specs.append(pl.BlockSpec((blk, d_dst), lambda i: (i, 0)))
    in_specs.append(pl.BlockSpec((d_dst, d_out), lambda i: (0, 0)))
    in_specs.append(pl.BlockSpec((8, d_out), lambda i: (0, 0)))
    out_specs = pl.BlockSpec((blk, d_out), lambda i: (i, 0))

    def body(*refs):
        idx = 0
        acc = None
        for (n_chunks, Wc) in chunk_lists:
            s_refs = refs[idx:idx + n_chunks]; idx += n_chunks
            cnt_ref = refs[idx]; idx += 1
            wl_refs = refs[idx:idx + n_chunks]; idx += n_chunks
            cnt = cnt_ref[0, :, 0:1] + cnt_ref[1, :, 0:1]
            inv = 1.0 / jnp.maximum(cnt, 1.0)
            for s_ref, wl_ref in zip(s_refs, wl_refs):
                mean = (s_ref[0] + s_ref[1]) * inv
                d = jnp.dot(mean, wl_ref[...],
                            preferred_element_type=jnp.float32)
                acc = d if acc is None else acc + d
        x_ref = refs[idx]
        wr_ref = refs[idx + 1]
        b_ref = refs[idx + 2]
        o_ref = refs[idx + 3]
        acc = acc + jnp.dot(x_ref[...], wr_ref[...],
                            preferred_element_type=jnp.float32) + b_ref[0:1, :]
        if relu:
            acc = jnp.maximum(acc, 0.0)
        o_ref[...] = acc

    return pl.pallas_call(
        body, grid=grid, in_specs=in_specs, out_specs=out_specs,
        out_shape=jax.ShapeDtypeStruct((n_pad, d_out), jnp.float32))


def _prep_edges(ei, n_dst):
    e = ei.shape[1]
    s_steps = _round_up(_round_up(e, NW * K) // (NW * K), SB)
    e_pad = NW * s_steps * K
    src = jnp.pad(ei[0], (0, e_pad - e), constant_values=0)
    dst = jnp.pad(ei[1], (0, e_pad - e), constant_values=n_dst)
    return (src.reshape(NW, s_steps, K), dst.reshape(NW, s_steps, K), s_steps)


def _chunks(x, n_chunks):
    wc = x.shape[1] // n_chunks
    return [x[:, c * wc:(c + 1) * wc] for c in range(n_chunks)]


def _row_chunks(x, n_chunks):
    wc = x.shape[0] // n_chunks
    return [x[c * wc:(c + 1) * wc, :] for c in range(n_chunks)]


def kernel(x_drug, x_protein, edge_index_ddi, edge_index_targets,
           edge_index_rev_targets, params):
    p = params
    nd, din = x_drug.shape
    npr, pin = x_protein.shape
    h = p['l1_ddi_Wl'].shape[0]
    d_out = p['l3_ddi_Wl'].shape[0]

    nd_pad = _round_up(nd, NS * ZB * 2)
    np_pad = _round_up(npr, NS * ZB * 2)
    xd = jnp.pad(x_drug, ((0, nd_pad - nd), (0, 0)))
    xp = jnp.pad(x_protein, ((0, np_pad - npr), (0, 0)))

    src_ddi, dst_ddi, s_ddi = _prep_edges(edge_index_ddi, nd)
    src_t, dst_t, s_t = _prep_edges(edge_index_targets, npr)
    src_rt, dst_rt, s_rt = _prep_edges(edge_index_rev_targets, nd)

    cnt_ddi_k = _make_cnt(nd_pad, s_ddi)
    cnt_t_k = _make_cnt(np_pad, s_t)
    cnt_rt_k = _make_cnt(nd_pad, s_rt)
    agg_ddi_k = _make_agg(nd_pad, nd_pad, s_ddi, 32, 4)
    agg_rt1_k = _make_agg(np_pad, nd_pad, s_rt, pin, 1)
    agg_rt_k = _make_agg(np_pad, nd_pad, s_rt, 32, 4)
    agg_t_k = _make_agg(nd_pad, np_pad, s_t, 64, 2)
    agg_ddi3_k = _make_agg(nd_pad, nd_pad, s_ddi, 32, 2)
    agg_rt3_k = _make_agg(np_pad, nd_pad, s_rt, 32, 2)
    agg_t3_k = _make_agg(nd_pad, np_pad, s_t, d_out, 1)

    cnt_ddi = cnt_ddi_k(dst_ddi)
    cnt_t = cnt_t_k(dst_t)
    cnt_rt = cnt_rt_k(dst_rt)

    blk = 1024
    upd_d1 = _make_update(nd_pad, blk, din, h, True, [(4, 32), (1, pin)])
    upd_p1 = _make_update(np_pad, blk, pin, h, True, [(2, 64)])
    upd_d2 = _make_update(nd_pad, blk, h, h, True, [(4, 32), (4, 32)])
    upd_p2 = _make_update(np_pad, blk, h, h, True, [(2, 64)])
    upd_d3 = _make_update(nd_pad, blk, h, d_out, False, [(2, 32), (2, 32)])
    upd_p3 = _make_update(np_pad, blk, h, d_out, False, [(1, d_out)])
    mm_d = _make_matmul(nd_pad, blk, h, d_out)
    mm_p = _make_matmul(np_pad, blk, h, d_out)

    def wT(name):
        return p[name].T

    def bias8(b):
        return jnp.broadcast_to(b.reshape(1, -1), (8, b.shape[0]))

    def as_list(x):
        return list(x) if isinstance(x, (list, tuple)) else [x]

    def drug_layer(upd, hd, hp, pre, agg_rt_kern, rt_chunks):
        s_ddi_parts = as_list(agg_ddi_k(*_chunks(hd, 4), src_ddi, dst_ddi))
        s_rt_parts = as_list(agg_rt_kern(*_chunks(hp, rt_chunks),
                                         src_rt, dst_rt))
        wl_ddi = _row_chunks(wT(pre + '_ddi_Wl'), 4)
        wl_rt = _row_chunks(wT(pre + '_rt_Wl'), rt_chunks)
        wr = wT(pre + '_ddi_Wr') + wT(pre + '_rt_Wr')
        b = bias8(p[pre + '_ddi_bl'] + p[pre + '_rt_bl'])
        return upd(*s_ddi_parts, cnt_ddi, *wl_ddi,
                   *s_rt_parts, cnt_rt, *wl_rt, hd, wr, b)

    def prot_layer(upd, hd, hp, pre):
        s_t_parts = as_list(agg_t_k(*_chunks(hd, 2), src_t, dst_t))
        return upd(*s_t_parts, cnt_t, *_row_chunks(wT(pre + '_t_Wl'), 2), hp,
                   wT(pre + '_t_Wr'), bias8(p[pre + '_t_bl']))

    hd1 = drug_layer(upd_d1, xd, xp, 'l1', agg_rt1_k, 1)
    hp1 = prot_layer(upd_p1, xd, xp, 'l1')
    hd2 = drug_layer(upd_d2, hd1, hp1, 'l2', agg_rt_k, 4)
    hp2 = prot_layer(upd_p2, hd1, hp1, 'l2')

    y_ddi = mm_d(hd2, wT('l3_ddi_Wl'))
    y_rt = mm_p(hp2, wT('l3_rt_Wl'))
    y_t = mm_d(hd2, wT('l3_t_Wl'))
    s_ddi3 = as_list(agg_ddi3_k(*_chunks(y_ddi, 2), src_ddi, dst_ddi))
    s_rt3 = as_list(agg_rt3_k(*_chunks(y_rt, 2), src_rt, dst_rt))
    s_t3 = as_list(agg_t3_k(y_t, src_t, dst_t))
    eye = jnp.eye(d_out, dtype=jnp.float32)
    od = upd_d3(*s_ddi3, cnt_ddi, *_row_chunks(eye, 2),
                *s_rt3, cnt_rt, *_row_chunks(eye, 2),
                hd2, wT('l3_ddi_Wr') + wT('l3_rt_Wr'),
                bias8(p['l3_ddi_bl'] + p['l3_rt_bl']))
    op = upd_p3(s_t3[0], cnt_t, eye, hp2, wT('l3_t_Wr'),
                bias8(p['l3_t_bl']))

    return od[:nd], op[:npr]


# speedup vs baseline: 1.7601x; 1.0150x over previous
import functools
import jax
import jax.numpy as jnp
from jax import lax
from jax.experimental import pallas as pl
from jax.experimental.pallas import tpu as pltpu
from jax.experimental.pallas import tpu_sc as plsc

NC = 2
NS = 16
NW = NC * NS
K = 128
SB = 16
ZB = 64


def _round_up(n, m):
    return (n + m - 1) // m * m


def _make_agg(n_src_pad, n_dst_pad, S, Wc, n_chunks):
    rows_per_tile = n_dst_pad // NS
    nz = rows_per_tile // ZB
    nb = S // SB
    nbuf = 4
    mesh = plsc.VectorSubcoreMesh(core_axis_name="c", subcore_axis_name="s")
    out_type = [jax.ShapeDtypeStruct((NC, n_dst_pad, Wc), jnp.float32)
                for _ in range(n_chunks)]
    scratch = [
        pltpu.VMEM((SB, K), jnp.int32),
        pltpu.VMEM((SB, K), jnp.int32),
        pltpu.VMEM((nbuf, K, Wc), jnp.float32),
        pltpu.VMEM((ZB, Wc), jnp.float32),
    ] + [pltpu.SemaphoreType.DMA] * nbuf + [
        pltpu.VMEM_SHARED((n_dst_pad, Wc), jnp.float32),
    ]

    def body(*refs):
        xs = refs[:n_chunks]
        src_hbm = refs[n_chunks]
        dst_hbm = refs[n_chunks + 1]
        outs = refs[n_chunks + 2: 2 * n_chunks + 2]
        rest = refs[2 * n_chunks + 2:]
        src_v, dst_v, rows_v, zero_v = rest[:4]
        sems = rest[4:4 + nbuf]
        acc = rest[4 + nbuf]
        cid = lax.axis_index("c")
        sid = lax.axis_index("s")
        w = cid * NS + sid

        zeros16 = jnp.zeros((16,), jnp.float32)

        def zfill(i, carry):
            for k in range(Wc // 16):
                zero_v[i, pl.ds(k * 16, 16)] = zeros16
            return carry
        lax.fori_loop(0, ZB, zfill, 0)

        tbase = sid * rows_per_tile
        for c in range(n_chunks):
            zdescs = [
                pltpu.async_copy(zero_v, acc.at[pl.ds(tbase + i * ZB, ZB)],
                                 sems[i % nbuf])
                for i in range(nz)
            ]
            for d in zdescs:
                d.wait()
            plsc.subcore_barrier()

            x_hbm = xs[c]

            def blk(bi, carry):
                pltpu.sync_copy(src_hbm.at[w, pl.ds(bi * SB, SB)], src_v)
                pltpu.sync_copy(dst_hbm.at[w, pl.ds(bi * SB, SB)], dst_v)
                descs = [
                    pltpu.async_copy(x_hbm.at[src_v.at[j]], rows_v.at[j],
                                     sems[j])
                    for j in range(nbuf)
                ]
                for s in range(SB):
                    j = s % nbuf
                    descs[j].wait()
                    pltpu.sync_copy(rows_v.at[j], acc.at[dst_v.at[s]],
                                    add=True)
                    if s + nbuf < SB:
                        descs[j] = pltpu.async_copy(
                            x_hbm.at[src_v.at[s + nbuf]], rows_v.at[j],
                            sems[j])
                return carry
            lax.fori_loop(0, nb, blk, 0)
            plsc.subcore_barrier()

            pltpu.sync_copy(acc.at[pl.ds(tbase, rows_per_tile)],
                            outs[c].at[cid, pl.ds(tbase, rows_per_tile)])

    return pl.kernel(body, out_type=out_type, mesh=mesh, scratch_types=scratch,
                     compiler_params=pltpu.CompilerParams(
                         use_tc_tiling_on_sc=False))


def _make_cnt(n_dst_pad, S):
    Wc = 16
    rows_per_tile = n_dst_pad // NS
    nz = rows_per_tile // ZB
    nb = S // SB
    mesh = plsc.VectorSubcoreMesh(core_axis_name="c", subcore_axis_name="s")
    out_type = jax.ShapeDtypeStruct((NC, n_dst_pad, Wc), jnp.float32)
    scratch = [
        pltpu.VMEM((SB, K), jnp.int32),
        pltpu.VMEM((K, Wc), jnp.float32),
        pltpu.VMEM((ZB, Wc), jnp.float32),
        pltpu.SemaphoreType.DMA,
        pltpu.VMEM_SHARED((n_dst_pad, Wc), jnp.float32),
    ]

    def body(dst_hbm, out_hbm, dst_v, ones_v, zero_v, zsem, acc):
        cid = lax.axis_index("c")
        sid = lax.axis_index("s")
        w = cid * NS + sid

        ones16 = jnp.ones((16,), jnp.float32)
        zeros16 = jnp.zeros((16,), jnp.float32)

        def fill(i, carry):
            zero_v[i % ZB, pl.ds(0, 16)] = zeros16
            ones_v[i, pl.ds(0, 16)] = ones16
            return carry
        lax.fori_loop(0, K, fill, 0)

        tbase = sid * rows_per_tile
        zdescs = [
            pltpu.async_copy(zero_v, acc.at[pl.ds(tbase + i * ZB, ZB)], zsem)
            for i in range(nz)
        ]
        for d in zdescs:
            d.wait()
        plsc.subcore_barrier()

        def blk(bi, carry):
            pltpu.sync_copy(dst_hbm.at[w, pl.ds(bi * SB, SB)], dst_v)

            def step(s, carry2):
                pltpu.sync_copy(ones_v, acc.at[dst_v.at[s]], add=True)
                return carry2
            lax.fori_loop(0, SB, step, 0)
            return carry
        lax.fori_loop(0, nb, blk, 0)
        plsc.subcore_barrier()

        pltpu.sync_copy(acc.at[pl.ds(tbase, rows_per_tile)],
                        out_hbm.at[cid, pl.ds(tbase, rows_per_tile)])

    return pl.kernel(body, out_type=out_type, mesh=mesh, scratch_types=scratch,
                     compiler_params=pltpu.CompilerParams(
                         use_tc_tiling_on_sc=False))


def _make_matmul(n_pad, blk, d_in, d_out):
    def body(x_ref, w_ref, o_ref):
        o_ref[...] = jnp.dot(x_ref[...], w_ref[...],
                             preferred_element_type=jnp.float32)

    return pl.pallas_call(
        body, grid=(n_pad // blk,),
        in_specs=[pl.BlockSpec((blk, d_in), lambda i: (i, 0)),
                  pl.BlockSpec((d_in, d_out), lambda i: (0, 0))],
        out_specs=pl.BlockSpec((blk, d_out), lambda i: (i, 0)),
        out_shape=jax.ShapeDtypeStruct((n_pad, d_out), jnp.float32))


def _make_update(n_pad, blk, d_dst, d_out, relu, chunk_lists):
    grid = (n_pad // blk,)
    in_specs = []
    for (n_chunks, Wc) in chunk_lists:
        for _ in range(n_chunks):
            in_specs.append(pl.BlockSpec((NC, blk, Wc), lambda i: (0, i, 0)))
        in_specs.append(pl.BlockSpec((NC, blk, 16), lambda i: (0, i, 0)))
        for _ in range(n_chunks):
            in_specs.append(pl.BlockSpec((Wc, d_out), lambda i: (0, 0)))
    in_specs.append(pl.BlockSpec((blk, d_dst), lambda i: (i, 0)))
    in_specs.append(pl.BlockSpec((d_dst, d_out), lambda i: (0, 0)))
    in_specs.append(pl.BlockSpec((8, d_out), lambda i: (0, 0)))
    out_specs = pl.BlockSpec((blk, d_out), lambda i: (i, 0))

    def body(*refs):
        idx = 0
        acc = None
        for (n_chunks, Wc) in chunk_lists:
            s_refs = refs[idx:idx + n_chunks]; idx += n_chunks
            cnt_ref = refs[idx]; idx += 1
            wl_refs = refs[idx:idx + n_chunks]; idx += n_chunks
            cnt = cnt_ref[0, :, 0:1] + cnt_ref[1, :, 0:1]
            inv = 1.0 / jnp.maximum(cnt, 1.0)
            for s_ref, wl_ref in zip(s_refs, wl_refs):
                mean = (s_ref[0] + s_ref[1]) * inv
                d = jnp.dot(mean, wl_ref[...],
                            preferred_element_type=jnp.float32)
                acc = d if acc is None else acc + d
        x_ref = refs[idx]
        wr_ref = refs[idx + 1]
        b_ref = refs[idx + 2]
        o_ref = refs[idx + 3]
        acc = acc + jnp.dot(x_ref[...], wr_ref[...],
                            preferred_element_type=jnp.float32) + b_ref[0:1, :]
        if relu:
            acc = jnp.maximum(acc, 0.0)
        o_ref[...] = acc

    return pl.pallas_call(
        body, grid=grid, in_specs=in_specs, out_specs=out_specs,
        out_shape=jax.ShapeDtypeStruct((n_pad, d_out), jnp.float32))


def _prep_edges(ei, n_dst):
    e = ei.shape[1]
    s_steps = _round_up(_round_up(e, NW * K) // (NW * K), SB)
    e_pad = NW * s_steps * K
    src = jnp.pad(ei[0], (0, e_pad - e), constant_values=0)
    dst = jnp.pad(ei[1], (0, e_pad - e), constant_values=n_dst)
    return (src.reshape(NW, s_steps, K), dst.reshape(NW, s_steps, K), s_steps)


def _chunks(x, n_chunks):
    wc = x.shape[1] // n_chunks
    return [x[:, c * wc:(c + 1) * wc] for c in range(n_chunks)]


def _row_chunks(x, n_chunks):
    wc = x.shape[0] // n_chunks
    return [x[c * wc:(c + 1) * wc, :] for c in range(n_chunks)]


def kernel(x_drug, x_protein, edge_index_ddi, edge_index_targets,
           edge_index_rev_targets, params):
    p = params
    nd, din = x_drug.shape
    npr, pin = x_protein.shape
    h = p['l1_ddi_Wl'].shape[0]
    d_out = p['l3_ddi_Wl'].shape[0]

    nd_pad = _round_up(nd, NS * ZB * 2)
    np_pad = _round_up(npr, NS * ZB * 2)
    xd = jnp.pad(x_drug, ((0, nd_pad - nd), (0, 0)))
    xp = jnp.pad(x_protein, ((0, np_pad - npr), (0, 0)))

    src_ddi, dst_ddi, s_ddi = _prep_edges(edge_index_ddi, nd)
    src_t, dst_t, s_t = _prep_edges(edge_index_targets, npr)
    src_rt, dst_rt, s_rt = _prep_edges(edge_index_rev_targets, nd)

    cnt_ddi_k = _make_cnt(nd_pad, s_ddi)
    cnt_t_k = _make_cnt(np_pad, s_t)
    cnt_rt_k = _make_cnt(nd_pad, s_rt)
    agg_ddi_k = _make_agg(nd_pad, nd_pad, s_ddi, 32, 4)
    agg_rt1_k = _make_agg(np_pad, nd_pad, s_rt, pin, 1)
    agg_rt_k = _make_agg(np_pad, nd_pad, s_rt, 32, 4)
    agg_t_k = _make_agg(nd_pad, np_pad, s_t, 64, 2)
    agg_ddi3_k = _make_agg(nd_pad, nd_pad, s_ddi, 32, 2)
    agg_rt3_k = _make_agg(np_pad, nd_pad, s_rt, 32, 2)
    agg_t3_k = _make_agg(nd_pad, np_pad, s_t, d_out, 1)

    cnt_ddi = cnt_ddi_k(dst_ddi)
    cnt_t = cnt_t_k(dst_t)
    cnt_rt = cnt_rt_k(dst_rt)

    blk = 1024
    upd_d1 = _make_update(nd_pad, blk, din, h, True, [(4, 32), (1, pin)])
    upd_p1 = _make_update(np_pad, blk, pin, h, True, [(2, 64)])
    upd_d2 = _make_update(nd_pad, blk, h, h, True, [(4, 32), (4, 32)])
    upd_p2 = _make_update(np_pad, blk, h, h, True, [(2, 64)])
    upd_d3 = _make_update(nd_pad, blk, h, d_out, False, [(2, 32), (2, 32)])
    upd_p3 = _make_update(np_pad, blk, h, d_out, False, [(1, d_out)])
    mm_d = _make_matmul(nd_pad, blk, h, d_out)
    mm_p = _make_matmul(np_pad, blk, h, d_out)

    def wT(name):
        return p[name].T

    def bias8(b):
        return jnp.broadcast_to(b.reshape(1, -1), (8, b.shape[0]))

    def as_list(x):
        return list(x) if isinstance(x, (list, tuple)) else [x]

    def drug_layer(upd, hd, hp, pre, agg_rt_kern, rt_chunks):
        s_ddi_parts = as_list(agg_ddi_k(*_chunks(hd, 4), src_ddi, dst_ddi))
        s_rt_parts = as_list(agg_rt_kern(*_chunks(hp, rt_chunks),
                                         src_rt, dst_rt))
        wl_ddi = _row_chunks(wT(pre + '_ddi_Wl'), 4)
        wl_rt = _row_chunks(wT(pre + '_rt_Wl'), rt_chunks)
        wr = wT(pre + '_ddi_Wr') + wT(pre + '_rt_Wr')
        b = bias8(p[pre + '_ddi_bl'] + p[pre + '_rt_bl'])
        return upd(*s_ddi_parts, cnt_ddi, *wl_ddi,
                   *s_rt_parts, cnt_rt, *wl_rt, hd, wr, b)

    def prot_layer(upd, hd, hp, pre):
        s_t_parts = as_list(agg_t_k(*_chunks(hd, 2), src_t, dst_t))
        return upd(*s_t_parts, cnt_t, *_row_chunks(wT(pre + '_t_Wl'), 2), hp,
                   wT(pre + '_t_Wr'), bias8(p[pre + '_t_bl']))

    hd1 = drug_layer(upd_d1, xd, xp, 'l1', agg_rt1_k, 1)
    hp1 = prot_layer(upd_p1, xd, xp, 'l1')
    hd2 = drug_layer(upd_d2, hd1, hp1, 'l2', agg_rt_k, 4)
    hp2 = prot_layer(upd_p2, hd1, hp1, 'l2')

    y_ddi = mm_d(hd2, wT('l3_ddi_Wl'))
    y_rt = mm_p(hp2, wT('l3_rt_Wl'))
    y_t = mm_d(hd2, wT('l3_t_Wl'))
    s_ddi3 = as_list(agg_ddi3_k(*_chunks(y_ddi, 2), src_ddi, dst_ddi))
    s_rt3 = as_list(agg_rt3_k(*_chunks(y_rt, 2), src_rt, dst_rt))
    s_t3 = as_list(agg_t3_k(y_t, src_t, dst_t))
    eye = jnp.eye(d_out, dtype=jnp.float32)
    od = upd_d3(*s_ddi3, cnt_ddi, *_row_chunks(eye, 2),
                *s_rt3, cnt_rt, *_row_chunks(eye, 2),
                hd2, wT('l3_ddi_Wr') + wT('l3_rt_Wr'),
                bias8(p['l3_ddi_bl'] + p['l3_rt_bl']))
    op = upd_p3(s_t3[0], cnt_t, eye, hp2, wT('l3_t_Wr'),
                bias8(p['l3_t_bl']))

    return od[:nd], op[:npr]


# speedup vs baseline: 1.8298x; 1.0396x over previous
import functools
import jax
import jax.numpy as jnp
from jax import lax
from jax.experimental import pallas as pl
from jax.experimental.pallas import tpu as pltpu
from jax.experimental.pallas import tpu_sc as plsc

NC = 2
NS = 16
NW = NC * NS
K = 128
SB = 16
ZB = 64


def _round_up(n, m):
    return (n + m - 1) // m * m


def _make_agg(n_src_pad, n_dst_pad, S, Wc, n_chunks, nbuf=4):
    rows_per_tile = n_dst_pad // NS
    nz = rows_per_tile // ZB
    nb = S // SB
    mesh = plsc.VectorSubcoreMesh(core_axis_name="c", subcore_axis_name="s")
    out_type = [jax.ShapeDtypeStruct((NC, n_dst_pad, Wc), jnp.float32)
                for _ in range(n_chunks)]
    scratch = [
        pltpu.VMEM((SB, K), jnp.int32),
        pltpu.VMEM((SB, K), jnp.int32),
        pltpu.VMEM((nbuf, K, Wc), jnp.float32),
        pltpu.VMEM((ZB, Wc), jnp.float32),
    ] + [pltpu.SemaphoreType.DMA] * nbuf + [
        pltpu.VMEM_SHARED((n_dst_pad, Wc), jnp.float32),
    ]

    def body(*refs):
        xs = refs[:n_chunks]
        src_hbm = refs[n_chunks]
        dst_hbm = refs[n_chunks + 1]
        outs = refs[n_chunks + 2: 2 * n_chunks + 2]
        rest = refs[2 * n_chunks + 2:]
        src_v, dst_v, rows_v, zero_v = rest[:4]
        sems = rest[4:4 + nbuf]
        acc = rest[4 + nbuf]
        cid = lax.axis_index("c")
        sid = lax.axis_index("s")
        w = cid * NS + sid

        zeros16 = jnp.zeros((16,), jnp.float32)

        def zfill(i, carry):
            for k in range(Wc // 16):
                zero_v[i, pl.ds(k * 16, 16)] = zeros16
            return carry
        lax.fori_loop(0, ZB, zfill, 0)

        tbase = sid * rows_per_tile
        for c in range(n_chunks):
            zdescs = [
                pltpu.async_copy(zero_v, acc.at[pl.ds(tbase + i * ZB, ZB)],
                                 sems[i % nbuf])
                for i in range(nz)
            ]
            for d in zdescs:
                d.wait()
            plsc.subcore_barrier()

            x_hbm = xs[c]

            def blk(bi, carry):
                pltpu.sync_copy(src_hbm.at[w, pl.ds(bi * SB, SB)], src_v)
                pltpu.sync_copy(dst_hbm.at[w, pl.ds(bi * SB, SB)], dst_v)
                descs = [
                    pltpu.async_copy(x_hbm.at[src_v.at[j]], rows_v.at[j],
                                     sems[j])
                    for j in range(nbuf)
                ]
                for s in range(SB):
                    j = s % nbuf
                    descs[j].wait()
                    pltpu.sync_copy(rows_v.at[j], acc.at[dst_v.at[s]],
                                    add=True)
                    if s + nbuf < SB:
                        descs[j] = pltpu.async_copy(
                            x_hbm.at[src_v.at[s + nbuf]], rows_v.at[j],
                            sems[j])
                return carry
            lax.fori_loop(0, nb, blk, 0)
            plsc.subcore_barrier()

            pltpu.sync_copy(acc.at[pl.ds(tbase, rows_per_tile)],
                            outs[c].at[cid, pl.ds(tbase, rows_per_tile)])

    return pl.kernel(body, out_type=out_type, mesh=mesh, scratch_types=scratch,
                     compiler_params=pltpu.CompilerParams(
                         use_tc_tiling_on_sc=False))


def _make_cnt(n_dst_pad, S):
    nb = S // SB
    mesh = plsc.VectorSubcoreMesh(core_axis_name="c", subcore_axis_name="s")
    out_type = jax.ShapeDtypeStruct((NC, NS, n_dst_pad), jnp.float32)
    scratch = [
        pltpu.VMEM((SB, K), jnp.int32),
        pltpu.VMEM((n_dst_pad,), jnp.float32),
    ]

    def body(dst_hbm, out_hbm, dst_v, cnt_v):
        cid = lax.axis_index("c")
        sid = lax.axis_index("s")
        w = cid * NS + sid

        ones16 = jnp.ones((16,), jnp.float32)
        zeros16 = jnp.zeros((16,), jnp.float32)

        def zf(i, carry):
            cnt_v[pl.ds(i * 16, 16)] = zeros16
            return carry
        lax.fori_loop(0, n_dst_pad // 16, zf, 0)

        def blk(bi, carry):
            pltpu.sync_copy(dst_hbm.at[w, pl.ds(bi * SB, SB)], dst_v)

            def step(s, carry2):
                for k in range(K // 16):
                    idx = dst_v[s, pl.ds(k * 16, 16)]
                    plsc.addupdate_scatter(cnt_v, [idx], ones16)
                return carry2
            lax.fori_loop(0, SB, step, 0)
            return carry
        lax.fori_loop(0, nb, blk, 0)

        pltpu.sync_copy(cnt_v, out_hbm.at[cid, sid])

    return pl.kernel(body, out_type=out_type, mesh=mesh, scratch_types=scratch,
                     compiler_params=pltpu.CompilerParams(
                         use_tc_tiling_on_sc=False,
                         needs_layout_passes=False))


def _make_matmul(n_pad, blk, d_in, d_out):
    def body(x_ref, w_ref, o_ref):
        o_ref[...] = jnp.dot(x_ref[...], w_ref[...],
                             preferred_element_type=jnp.float32)

    return pl.pallas_call(
        body, grid=(n_pad // blk,),
        in_specs=[pl.BlockSpec((blk, d_in), lambda i: (i, 0)),
                  pl.BlockSpec((d_in, d_out), lambda i: (0, 0))],
        out_specs=pl.BlockSpec((blk, d_out), lambda i: (i, 0)),
        out_shape=jax.ShapeDtypeStruct((n_pad, d_out), jnp.float32))


def _make_update(n_pad, blk, d_dst, d_out, relu, chunk_lists):
    grid = (n_pad // blk,)
    in_specs = []
    for (n_chunks, Wc) in chunk_lists:
        for _ in range(n_chunks):
            in_specs.append(pl.BlockSpec((NC, blk, Wc), lambda i: (0, i, 0)))
        in_specs.append(pl.BlockSpec((NC, NS, blk), lambda i: (0, 0, i)))
        for _ in range(n_chunks):
            in_specs.append(pl.BlockSpec((Wc, d_out), lambda i: (0, 0)))
    in_specs.append(pl.BlockSpec((blk, d_dst), lambda i: (i, 0)))
    in_specs.append(pl.BlockSpec((d_dst, d_out), lambda i: (0, 0)))
    in_specs.append(pl.BlockSpec((8, d_out), lambda i: (0, 0)))
    out_specs = pl.BlockSpec((blk, d_out), lambda i: (i, 0))

    def body(*refs):
        idx = 0
        acc = None
        for (n_chunks, Wc) in chunk_lists:
            s_refs = refs[idx:idx + n_chunks]; idx += n_chunks
            cnt_ref = refs[idx]; idx += 1
            wl_refs = refs[idx:idx + n_chunks]; idx += n_chunks
            cnt = jnp.sum(cnt_ref[...], axis=(0, 1))[:, None]
            inv = 1.0 / jnp.maximum(cnt, 1.0)
            for s_ref, wl_ref in zip(s_refs, wl_refs):
                mean = (s_ref[0] + s_ref[1]) * inv
                d = jnp.dot(mean, wl_ref[...],
                            preferred_element_type=jnp.float32)
                acc = d if acc is None else acc + d
        x_ref = refs[idx]
        wr_ref = refs[idx + 1]
        b_ref = refs[idx + 2]
        o_ref = refs[idx + 3]
        acc = acc + jnp.dot(x_ref[...], wr_ref[...],
                            preferred_element_type=jnp.float32) + b_ref[0:1, :]
        if relu:
            acc = jnp.maximum(acc, 0.0)
        o_ref[...] = acc

    return pl.pallas_call(
        body, grid=grid, in_specs=in_specs, out_specs=out_specs,
        out_shape=jax.ShapeDtypeStruct((n_pad, d_out), jnp.float32))


def _prep_edges(ei, n_dst):
    e = ei.shape[1]
    s_steps = _round_up(_round_up(e, NW * K) // (NW * K), SB)
    e_pad = NW * s_steps * K
    src = jnp.pad(ei[0], (0, e_pad - e), constant_values=0)
    dst = jnp.pad(ei[1], (0, e_pad - e), constant_values=n_dst)
    return (src.reshape(NW, s_steps, K), dst.reshape(NW, s_steps, K), s_steps)


def _chunks(x, n_chunks):
    wc = x.shape[1] // n_chunks
    return [x[:, c * wc:(c + 1) * wc] for c in range(n_chunks)]


def _row_chunks(x, n_chunks):
    wc = x.shape[0] // n_chunks
    return [x[c * wc:(c + 1) * wc, :] for c in range(n_chunks)]


def kernel(x_drug, x_protein, edge_index_ddi, edge_index_targets,
           edge_index_rev_targets, params):
    p = params
    nd, din = x_drug.shape
    npr, pin = x_protein.shape
    h = p['l1_ddi_Wl'].shape[0]
    d_out = p['l3_ddi_Wl'].shape[0]

    nd_pad = _round_up(nd, NS * ZB * 2)
    np_pad = _round_up(npr, NS * ZB * 2)
    xd = jnp.pad(x_drug, ((0, nd_pad - nd), (0, 0)))
    xp = jnp.pad(x_protein, ((0, np_pad - npr), (0, 0)))

    src_ddi, dst_ddi, s_ddi = _prep_edges(edge_index_ddi, nd)
    src_t, dst_t, s_t = _prep_edges(edge_index_targets, npr)
    src_rt, dst_rt, s_rt = _prep_edges(edge_index_rev_targets, nd)

    cnt_ddi_k = _make_cnt(nd_pad, s_ddi)
    cnt_t_k = _make_cnt(np_pad, s_t)
    cnt_rt_k = _make_cnt(nd_pad, s_rt)
    agg_ddi_k = _make_agg(nd_pad, nd_pad, s_ddi, 32, 4)
    agg_rt1_k = _make_agg(np_pad, nd_pad, s_rt, pin, 1)
    agg_rt_k = _make_agg(np_pad, nd_pad, s_rt, 32, 4)
    agg_t_k = _make_agg(nd_pad, np_pad, s_t, h, 1, nbuf=2)
    agg_ddi3_k = _make_agg(nd_pad, nd_pad, s_ddi, 32, 2)
    agg_rt3_k = _make_agg(np_pad, nd_pad, s_rt, 32, 2)
    agg_t3_k = _make_agg(nd_pad, np_pad, s_t, d_out, 1)

    cnt_ddi = cnt_ddi_k(dst_ddi)
    cnt_t = cnt_t_k(dst_t)
    cnt_rt = cnt_rt_k(dst_rt)

    blk = 1024
    upd_d1 = _make_update(nd_pad, blk, din, h, True, [(4, 32), (1, pin)])
    upd_p1 = _make_update(np_pad, blk, pin, h, True, [(1, h)])
    upd_d2 = _make_update(nd_pad, blk, h, h, True, [(4, 32), (4, 32)])
    upd_p2 = _make_update(np_pad, blk, h, h, True, [(1, h)])
    upd_d3 = _make_update(nd_pad, blk, h, d_out, False, [(2, 32), (2, 32)])
    upd_p3 = _make_update(np_pad, blk, h, d_out, False, [(1, d_out)])
    mm_d = _make_matmul(nd_pad, blk, h, d_out)
    mm_p = _make_matmul(np_pad, blk, h, d_out)

    def wT(name):
        return p[name].T

    def bias8(b):
        return jnp.broadcast_to(b.reshape(1, -1), (8, b.shape[0]))

    def as_list(x):
        return list(x) if isinstance(x, (list, tuple)) else [x]

    def drug_layer(upd, hd, hp, pre, agg_rt_kern, rt_chunks):
        s_ddi_parts = as_list(agg_ddi_k(*_chunks(hd, 4), src_ddi, dst_ddi))
        s_rt_parts = as_list(agg_rt_kern(*_chunks(hp, rt_chunks),
                                         src_rt, dst_rt))
        wl_ddi = _row_chunks(wT(pre + '_ddi_Wl'), 4)
        wl_rt = _row_chunks(wT(pre + '_rt_Wl'), rt_chunks)
        wr = wT(pre + '_ddi_Wr') + wT(pre + '_rt_Wr')
        b = bias8(p[pre + '_ddi_bl'] + p[pre + '_rt_bl'])
        return upd(*s_ddi_parts, cnt_ddi, *wl_ddi,
                   *s_rt_parts, cnt_rt, *wl_rt, hd, wr, b)

    def prot_layer(upd, hd, hp, pre):
        s_t_parts = as_list(agg_t_k(hd, src_t, dst_t))
        return upd(s_t_parts[0], cnt_t, wT(pre + '_t_Wl'), hp,
                   wT(pre + '_t_Wr'), bias8(p[pre + '_t_bl']))

    hd1 = drug_layer(upd_d1, xd, xp, 'l1', agg_rt1_k, 1)
    hp1 = prot_layer(upd_p1, xd, xp, 'l1')
    hd2 = drug_layer(upd_d2, hd1, hp1, 'l2', agg_rt_k, 4)
    hp2 = prot_layer(upd_p2, hd1, hp1, 'l2')

    y_ddi = mm_d(hd2, wT('l3_ddi_Wl'))
    y_rt = mm_p(hp2, wT('l3_rt_Wl'))
    y_t = mm_d(hd2, wT('l3_t_Wl'))
    s_ddi3 = as_list(agg_ddi3_k(*_chunks(y_ddi, 2), src_ddi, dst_ddi))
    s_rt3 = as_list(agg_rt3_k(*_chunks(y_rt, 2), src_rt, dst_rt))
    s_t3 = as_list(agg_t3_k(y_t, src_t, dst_t))
    eye = jnp.eye(d_out, dtype=jnp.float32)
    od = upd_d3(*s_ddi3, cnt_ddi, *_row_chunks(eye, 2),
                *s_rt3, cnt_rt, *_row_chunks(eye, 2),
                hd2, wT('l3_ddi_Wr') + wT('l3_rt_Wr'),
                bias8(p['l3_ddi_bl'] + p['l3_rt_bl']))
    op = upd_p3(s_t3[0], cnt_t, eye, hp2, wT('l3_t_Wr'),
                bias8(p['l3_t_bl']))

    return od[:nd], op[:npr]
